# baseline (speedup 1.0000x reference)
import jax
import jax.numpy as jnp
from jax.experimental import pallas as pl

_BN = 1000
_BE = 2000


def _ln_body(x_ref, g_ref, b_ref, o_ref):
    xv = x_ref[...]
    mu = jnp.mean(xv, axis=1, keepdims=True)
    var = jnp.mean((xv - mu) ** 2, axis=1, keepdims=True)
    o_ref[...] = (xv - mu) / jnp.sqrt(var + 1e-5) * g_ref[...] + b_ref[...]


def _edge_body(xs_ref, xr_ref, el_ref, rv_ref, tv_ref, acat_ref, b1c_ref,
               w2bd_ref, b2r_ref, bcat_ref, tb1c_ref, t2bd_ref, tb2r_ref,
               rdls_ref, rtb_ref, rtw_ref, mb_ref, ms_ref,
               lg_ref, mix_ref, dx_ref):
    dx = xs_ref[...] - xr_ref[...]
    dx_ref[...] = dx
    rlog = jnp.dot(dx, rv_ref[...], preferred_element_type=jnp.float32)
    tlog = jnp.dot(dx, tv_ref[...], preferred_element_type=jnp.float32)
    h1 = jax.nn.silu(jnp.dot(dx, acat_ref[...],
                             preferred_element_type=jnp.float32) + b1c_ref[...])
    dec = jax.nn.softplus(jnp.dot(h1, w2bd_ref[...],
                                  preferred_element_type=jnp.float32) + b2r_ref[...])
    h2 = jax.nn.silu(jnp.dot(dx, bcat_ref[...],
                             preferred_element_type=jnp.float32) + tb1c_ref[...])
    tm = jnp.dot(h2, t2bd_ref[...], preferred_element_type=jnp.float32) + tb2r_ref[...]
    temp = jax.nn.softplus(rtb_ref[...] + rtw_ref[...] * tm) + 0.1
    el = el_ref[...]
    mix = jax.nn.sigmoid(mb_ref[...] + ms_ref[...] * el)
    dscale = jax.nn.softplus(rdls_ref[...])
    ddlog = jnp.log(jnp.exp(-dec * dscale * el) + 1e-9)
    lg_ref[...] = (mix * rlog + (1.0 - mix) * tlog) / temp + ddlog
    mix_ref[...] = mix


def _msg_body(dx_ref, lg_ref, mr_ref, dr_ref, mix_ref, rw_ref, tw_ref, o_ref):
    alpha = jnp.exp(lg_ref[...] - mr_ref[...]) / (dr_ref[...] + 1e-9)
    mix = mix_ref[...]
    a = alpha * mix
    b = alpha * (1.0 - mix)
    dx = dx_ref[...]
    H = rw_ref.shape[0]
    acc = jnp.zeros(dx.shape, jnp.float32)
    for h in range(H):
        acc = acc + a[:, h:h + 1] * jnp.dot(dx, rw_ref[h],
                                            preferred_element_type=jnp.float32)
        acc = acc + b[:, h:h + 1] * jnp.dot(dx, tw_ref[h],
                                            preferred_element_type=jnp.float32)
    o_ref[...] = acc


def _out_body(agg_ref, xn_ref, wo_ref, ls_ref, o_ref):
    out = jnp.nan_to_num(agg_ref[...])
    out = jnp.dot(out, wo_ref[...], preferred_element_type=jnp.float32)
    o_ref[...] = xn_ref[...] + out * ls_ref[...]


def _full(shape):
    nd = len(shape)
    return pl.BlockSpec(shape, lambda i, _nd=nd: (0,) * _nd)


def kernel(x, edge_index, edge_vec, edge_len, w_proj, radial_w, tangential_w,
           w_out, ln_gamma, ln_beta, radial_score, tangential_score,
           radial_distance_log_scale, radial_temp_bias, radial_temp_weight,
           mix_bias, mix_scale, decay_w1, decay_b1, decay_w2, decay_b2,
           temp_w1, temp_b1, temp_w2, temp_b2, layer_scale):
    N, F = x.shape
    E = edge_index.shape[1]
    H, _, FM = decay_w1.shape
    inv = 1.0 / jnp.sqrt(jnp.float32(F))
    sender = edge_index[0]
    receiver = edge_index[1]

    rv = jnp.einsum('hfg,hg->fh', radial_w, radial_score) * (inv * inv)
    tv = jnp.einsum('hfg,hg->fh', tangential_w, tangential_score) * (inv * inv)
    acat = (jnp.einsum('hfg,hgm->hfm', w_proj, decay_w1) * inv
            ).transpose(1, 0, 2).reshape(F, H * FM)
    bcat = (jnp.einsum('hfg,hgm->hfm', w_proj, temp_w1) * inv
            ).transpose(1, 0, 2).reshape(F, H * FM)
    b1c = decay_b1.reshape(1, H * FM)
    tb1c = temp_b1.reshape(1, H * FM)
    eye = jnp.eye(H, dtype=jnp.float32)
    w2bd = (decay_w2[:, :, 0:1] * eye[:, None, :]).reshape(H * FM, H)
    t2bd = (temp_w2[:, :, 0:1] * eye[:, None, :]).reshape(H * FM, H)
    b2r = decay_b2[:, 0].reshape(1, H)
    tb2r = temp_b2[:, 0].reshape(1, H)
    rdls = radial_distance_log_scale.reshape(1, H)
    rtb = radial_temp_bias.reshape(1, H)
    rtw = radial_temp_weight.reshape(1, H)
    mb = mix_bias.reshape(1, H)
    ms = mix_scale.reshape(1, H)

    xn = pl.pallas_call(
        _ln_body,
        grid=(N // _BN,),
        in_specs=[pl.BlockSpec((_BN, F), lambda i: (i, 0)),
                  _full((1, F)), _full((1, F))],
        out_specs=pl.BlockSpec((_BN, F), lambda i: (i, 0)),
        out_shape=jax.ShapeDtypeStruct((N, F), jnp.float32),
    )(x, ln_gamma.reshape(1, F), ln_beta.reshape(1, F))

    xs = jnp.take(xn, sender, axis=0)
    xr = jnp.take(xn, receiver, axis=0)
    el2 = edge_len.reshape(E, 1)

    eb = pl.BlockSpec((_BE, F), lambda i: (i, 0))
    e4 = pl.BlockSpec((_BE, H), lambda i: (i, 0))
    e1 = pl.BlockSpec((_BE, 1), lambda i: (i, 0))
    logits, mix, dx = pl.pallas_call(
        _edge_body,
        grid=(E // _BE,),
        in_specs=[eb, eb, e1,
                  _full((F, H)), _full((F, H)),
                  _full((F, H * FM)), _full((1, H * FM)),
                  _full((H * FM, H)), _full((1, H)),
                  _full((F, H * FM)), _full((1, H * FM)),
                  _full((H * FM, H)), _full((1, H)),
                  _full((1, H)), _full((1, H)), _full((1, H)),
                  _full((1, H)), _full((1, H))],
        out_specs=[e4, e4, eb],
        out_shape=[jax.ShapeDtypeStruct((E, H), jnp.float32),
                   jax.ShapeDtypeStruct((E, H), jnp.float32),
                   jax.ShapeDtypeStruct((E, F), jnp.float32)],
    )(xs, xr, el2, rv, tv, acat, b1c, w2bd, b2r, bcat, tb1c, t2bd, tb2r,
      rdls, rtb, rtw, mb, ms)

    m = jax.ops.segment_max(logits, receiver, num_segments=N)
    den = jax.ops.segment_sum(jnp.exp(logits - m[receiver]), receiver,
                              num_segments=N)
    md = jnp.concatenate([m, den], axis=1)[receiver]
    mr = md[:, :H]
    dr = md[:, H:]

    msg = pl.pallas_call(
        _msg_body,
        grid=(E // _BE,),
        in_specs=[eb, e4, e4, e4, e4,
                  _full((H, F, F)), _full((H, F, F))],
        out_specs=eb,
        out_shape=jax.ShapeDtypeStruct((E, F), jnp.float32),
    )(dx, logits, mr, dr, mix, radial_w * (inv / H), tangential_w * (inv / H))

    agg = jax.ops.segment_sum(msg, receiver, num_segments=N)

    nb = pl.BlockSpec((_BN, F), lambda i: (i, 0))
    out = pl.pallas_call(
        _out_body,
        grid=(N // _BN,),
        in_specs=[nb, nb, _full((F, F)), _full((1, F))],
        out_specs=nb,
        out_shape=jax.ShapeDtypeStruct((N, F), jnp.float32),
    )(agg, xn, w_out * inv, layer_scale.reshape(1, F))
    return out


# speedup vs baseline: 1.0592x; 1.0592x over previous
import jax
import jax.numpy as jnp
from jax.experimental import pallas as pl

_BN = 1000
_BE = 4000


def _ln_body(x_ref, g_ref, b_ref, o_ref):
    xv = x_ref[...]
    mu = jnp.mean(xv, axis=1, keepdims=True)
    var = jnp.mean((xv - mu) ** 2, axis=1, keepdims=True)
    o_ref[...] = (xv - mu) / jnp.sqrt(var + 1e-5) * g_ref[...] + b_ref[...]


def _edge_body(xs_ref, xr_ref, el_ref, rv_ref, tv_ref, acat_ref, b1c_ref,
               w2bd_ref, b2r_ref, bcat_ref, tb1c_ref, t2bd_ref, tb2r_ref,
               rdls_ref, rtb_ref, rtw_ref, mb_ref, ms_ref,
               lg_ref, mix_ref, dx_ref):
    dx = xs_ref[...] - xr_ref[...]
    dx_ref[...] = dx
    rlog = jnp.dot(dx, rv_ref[...], preferred_element_type=jnp.float32)
    tlog = jnp.dot(dx, tv_ref[...], preferred_element_type=jnp.float32)
    h1 = jax.nn.silu(jnp.dot(dx, acat_ref[...],
                             preferred_element_type=jnp.float32) + b1c_ref[...])
    dec = jax.nn.softplus(jnp.dot(h1, w2bd_ref[...],
                                  preferred_element_type=jnp.float32) + b2r_ref[...])
    h2 = jax.nn.silu(jnp.dot(dx, bcat_ref[...],
                             preferred_element_type=jnp.float32) + tb1c_ref[...])
    tm = jnp.dot(h2, t2bd_ref[...], preferred_element_type=jnp.float32) + tb2r_ref[...]
    temp = jax.nn.softplus(rtb_ref[...] + rtw_ref[...] * tm) + 0.1
    el = el_ref[...]
    mix = jax.nn.sigmoid(mb_ref[...] + ms_ref[...] * el)
    dscale = jax.nn.softplus(rdls_ref[...])
    ddlog = jnp.log(jnp.exp(-dec * dscale * el) + 1e-9)
    lg_ref[...] = (mix * rlog + (1.0 - mix) * tlog) / temp + ddlog
    mix_ref[...] = mix


def _msg_body(dx_ref, lg_ref, mr_ref, dr_ref, mix_ref, rw_ref, tw_ref, o_ref):
    alpha = jnp.exp(lg_ref[...] - mr_ref[...]) / (dr_ref[...] + 1e-9)
    mix = mix_ref[...]
    a = alpha * mix
    b = alpha * (1.0 - mix)
    dx = dx_ref[...]
    H = rw_ref.shape[0]
    acc = jnp.zeros(dx.shape, jnp.float32)
    for h in range(H):
        acc = acc + a[:, h:h + 1] * jnp.dot(dx, rw_ref[h],
                                            preferred_element_type=jnp.float32)
        acc = acc + b[:, h:h + 1] * jnp.dot(dx, tw_ref[h],
                                            preferred_element_type=jnp.float32)
    o_ref[...] = acc


def _out_body(agg_ref, xn_ref, wo_ref, ls_ref, o_ref):
    out = jnp.nan_to_num(agg_ref[...])
    out = jnp.dot(out, wo_ref[...], preferred_element_type=jnp.float32)
    o_ref[...] = xn_ref[...] + out * ls_ref[...]


def _full(shape):
    nd = len(shape)
    return pl.BlockSpec(shape, lambda i, _nd=nd: (0,) * _nd)


def kernel(x, edge_index, edge_vec, edge_len, w_proj, radial_w, tangential_w,
           w_out, ln_gamma, ln_beta, radial_score, tangential_score,
           radial_distance_log_scale, radial_temp_bias, radial_temp_weight,
           mix_bias, mix_scale, decay_w1, decay_b1, decay_w2, decay_b2,
           temp_w1, temp_b1, temp_w2, temp_b2, layer_scale):
    N, F = x.shape
    E = edge_index.shape[1]
    H, _, FM = decay_w1.shape
    inv = 1.0 / jnp.sqrt(jnp.float32(F))
    sender = edge_index[0]
    receiver = edge_index[1]

    rv = jnp.einsum('hfg,hg->fh', radial_w, radial_score) * (inv * inv)
    tv = jnp.einsum('hfg,hg->fh', tangential_w, tangential_score) * (inv * inv)
    acat = (jnp.einsum('hfg,hgm->hfm', w_proj, decay_w1) * inv
            ).transpose(1, 0, 2).reshape(F, H * FM)
    bcat = (jnp.einsum('hfg,hgm->hfm', w_proj, temp_w1) * inv
            ).transpose(1, 0, 2).reshape(F, H * FM)
    b1c = decay_b1.reshape(1, H * FM)
    tb1c = temp_b1.reshape(1, H * FM)
    eye = jnp.eye(H, dtype=jnp.float32)
    w2bd = (decay_w2[:, :, 0:1] * eye[:, None, :]).reshape(H * FM, H)
    t2bd = (temp_w2[:, :, 0:1] * eye[:, None, :]).reshape(H * FM, H)
    b2r = decay_b2[:, 0].reshape(1, H)
    tb2r = temp_b2[:, 0].reshape(1, H)
    rdls = radial_distance_log_scale.reshape(1, H)
    rtb = radial_temp_bias.reshape(1, H)
    rtw = radial_temp_weight.reshape(1, H)
    mb = mix_bias.reshape(1, H)
    ms = mix_scale.reshape(1, H)

    xn = pl.pallas_call(
        _ln_body,
        grid=(N // _BN,),
        in_specs=[pl.BlockSpec((_BN, F), lambda i: (i, 0)),
                  _full((1, F)), _full((1, F))],
        out_specs=pl.BlockSpec((_BN, F), lambda i: (i, 0)),
        out_shape=jax.ShapeDtypeStruct((N, F), jnp.float32),
    )(x, ln_gamma.reshape(1, F), ln_beta.reshape(1, F))

    xs = jnp.take(xn, sender, axis=0)
    xr = jnp.take(xn, receiver, axis=0)
    el2 = edge_len.reshape(E, 1)

    eb = pl.BlockSpec((_BE, F), lambda i: (i, 0))
    e4 = pl.BlockSpec((_BE, H), lambda i: (i, 0))
    e1 = pl.BlockSpec((_BE, 1), lambda i: (i, 0))
    logits, mix, dx = pl.pallas_call(
        _edge_body,
        grid=(E // _BE,),
        in_specs=[eb, eb, e1,
                  _full((F, H)), _full((F, H)),
                  _full((F, H * FM)), _full((1, H * FM)),
                  _full((H * FM, H)), _full((1, H)),
                  _full((F, H * FM)), _full((1, H * FM)),
                  _full((H * FM, H)), _full((1, H)),
                  _full((1, H)), _full((1, H)), _full((1, H)),
                  _full((1, H)), _full((1, H))],
        out_specs=[e4, e4, eb],
        out_shape=[jax.ShapeDtypeStruct((E, H), jnp.float32),
                   jax.ShapeDtypeStruct((E, H), jnp.float32),
                   jax.ShapeDtypeStruct((E, F), jnp.float32)],
    )(xs, xr, el2, rv, tv, acat, b1c, w2bd, b2r, bcat, tb1c, t2bd, tb2r,
      rdls, rtb, rtw, mb, ms)

    m = jax.ops.segment_max(logits, receiver, num_segments=N)
    den = jax.ops.segment_sum(jnp.exp(logits - m[receiver]), receiver,
                              num_segments=N)
    mr = m[receiver]
    dr = den[receiver]

    msg = pl.pallas_call(
        _msg_body,
        grid=(E // _BE,),
        in_specs=[eb, e4, e4, e4, e4,
                  _full((H, F, F)), _full((H, F, F))],
        out_specs=eb,
        out_shape=jax.ShapeDtypeStruct((E, F), jnp.float32),
    )(dx, logits, mr, dr, mix, radial_w * (inv / H), tangential_w * (inv / H))

    agg = jax.ops.segment_sum(msg, receiver, num_segments=N)

    nb = pl.BlockSpec((_BN, F), lambda i: (i, 0))
    out = pl.pallas_call(
        _out_body,
        grid=(N // _BN,),
        in_specs=[nb, nb, _full((F, F)), _full((1, F))],
        out_specs=nb,
        out_shape=jax.ShapeDtypeStruct((N, F), jnp.float32),
    )(agg, xn, w_out * inv, layer_scale.reshape(1, F))
    return out
